# baseline (speedup 1.0000x reference)
import jax
import jax.numpy as jnp
from jax import lax
from jax.experimental import pallas as pl
from jax.experimental.pallas import tpu as pltpu
from jax.experimental.pallas import tpu_sc as plsc

NUM_ENTITIES = 100000
NUM_RELS = 1000
FEAT_DIM = 128
BATCH = 16384

NC = 2
NS = 16
L = 16
NW = NC * NS
ROWS_PER_W = BATCH // NW
CHUNK = 64
NCHUNKS = ROWS_PER_W // CHUNK
NBUF = 4
GROUPS = CHUNK // L
UNROLL = 16
NACC = 4


def _sc_body(embed_hbm, head_hbm, rel_hbm, tail_hbm, wrel_hbm, out_hbm,
             hidx_v, ridx_v, tidx_v, hbufs, rbufs, tbufs, out_v, sems):
    wid = lax.axis_index("s") * NC + lax.axis_index("c")
    base = wid * ROWS_PER_W

    with jax.named_scope("stage_idx"):
        cp_i = [
            pltpu.async_copy(head_hbm.at[pl.ds(base, ROWS_PER_W)], hidx_v, sems[0]),
            pltpu.async_copy(rel_hbm.at[pl.ds(base, ROWS_PER_W)], ridx_v, sems[1]),
            pltpu.async_copy(tail_hbm.at[pl.ds(base, ROWS_PER_W)], tidx_v, sems[2]),
        ]
        for cp in cp_i:
            cp.wait()

    lane = lax.broadcasted_iota(jnp.int32, (L,), 0)

    def issue(c):
        k = c % NBUF
        sl = pl.ds(c * CHUNK, CHUNK)
        return [
            pltpu.async_copy(embed_hbm.at[hidx_v.at[sl]], hbufs[k], sems[k]),
            pltpu.async_copy(wrel_hbm.at[ridx_v.at[sl]], rbufs[k], sems[k]),
            pltpu.async_copy(embed_hbm.at[tidx_v.at[sl]], tbufs[k], sems[k]),
        ]

    pending = {}
    for c in range(NBUF - 1):
        pending[c] = issue(c)

    for c in range(NCHUNKS):
        if c + NBUF - 1 < NCHUNKS:
            pending[c + NBUF - 1] = issue(c + NBUF - 1)
        with jax.named_scope("dma_wait%d" % c):
            for cp in pending.pop(c):
                cp.wait()
        k = c % NBUF
        h_v, r_v, t_v = hbufs[k], rbufs[k], tbufs[k]

        def group_body(g, _):
            rows = g * L + lane

            def dstep(_, carry):
                accs, col = carry
                accs = list(accs)
                for u in range(UNROLL):
                    hv = plsc.load_gather(h_v, [rows, col])
                    rv = plsc.load_gather(r_v, [rows, col])
                    tv = plsc.load_gather(t_v, [rows, col])
                    accs[u % NACC] = accs[u % NACC] + hv * rv * tv
                    col = (col + 1) & (FEAT_DIM - 1)
                return tuple(accs), col

            zero = jnp.zeros((L,), jnp.float32)
            accs, _ = lax.fori_loop(
                0, FEAT_DIM // UNROLL, dstep,
                ((zero,) * NACC, lane))
            acc = (accs[0] + accs[1]) + (accs[2] + accs[3])
            out_v[pl.ds(c * CHUNK + g * L, L)] = acc
            return 0

        with jax.named_scope("compute%d" % c):
            lax.fori_loop(0, GROUPS, group_body, 0)

    with jax.named_scope("drain_out"):
        pltpu.sync_copy(out_v, out_hbm.at[pl.ds(base, ROWS_PER_W)])


@jax.jit
def _dist_mult(embed, head, rel, tail, w_relations):
    mesh = plsc.VectorSubcoreMesh(core_axis_name="c", subcore_axis_name="s")
    rowbuf = pltpu.VMEM((CHUNK, FEAT_DIM), jnp.float32)
    run = pl.kernel(
        _sc_body,
        out_type=jax.ShapeDtypeStruct((BATCH,), jnp.float32),
        mesh=mesh,
        compiler_params=pltpu.CompilerParams(needs_layout_passes=False),
        scratch_types=[
            pltpu.VMEM((ROWS_PER_W,), jnp.int32),
            pltpu.VMEM((ROWS_PER_W,), jnp.int32),
            pltpu.VMEM((ROWS_PER_W,), jnp.int32),
            [rowbuf] * NBUF,
            [rowbuf] * NBUF,
            [rowbuf] * NBUF,
            pltpu.VMEM((ROWS_PER_W,), jnp.float32),
            [pltpu.SemaphoreType.DMA] * NBUF,
        ],
    )
    return run(embed, head, rel, tail, w_relations)


def kernel(embed, head, rel, tail, w_relations):
    head = head.astype(jnp.int32)
    rel = rel.astype(jnp.int32)
    tail = tail.astype(jnp.int32)
    return _dist_mult(embed, head, rel, tail, w_relations)


# speedup vs baseline: 1.0314x; 1.0314x over previous
import jax
import jax.numpy as jnp
from jax import lax
from jax.experimental import pallas as pl
from jax.experimental.pallas import tpu as pltpu
from jax.experimental.pallas import tpu_sc as plsc

NUM_ENTITIES = 100000
NUM_RELS = 1000
FEAT_DIM = 128
BATCH = 16384

NC = 2
NS = 16
L = 16
NW = NC * NS
ROWS_PER_W = BATCH // NW
CHUNK = 64
NCHUNKS = ROWS_PER_W // CHUNK
NBUF = 4
GROUPS = CHUNK // L
UNROLL = 16
NACC = 4


def _sc_body(embed_hbm, head_hbm, rel_hbm, tail_hbm, wrel_hbm, out_hbm,
             hidx_v, ridx_v, tidx_v, hbufs, rbufs, tbufs, out_v, wrel_sh, sems):
    sid = lax.axis_index("s")
    wid = sid * NC + lax.axis_index("c")
    base = wid * ROWS_PER_W

    @pl.when(sid == 0)
    def _():
        pltpu.sync_copy(wrel_hbm, wrel_sh)

    with jax.named_scope("stage_idx"):
        cp_i = [
            pltpu.async_copy(head_hbm.at[pl.ds(base, ROWS_PER_W)], hidx_v, sems[0]),
            pltpu.async_copy(rel_hbm.at[pl.ds(base, ROWS_PER_W)], ridx_v, sems[1]),
            pltpu.async_copy(tail_hbm.at[pl.ds(base, ROWS_PER_W)], tidx_v, sems[2]),
        ]
        for cp in cp_i:
            cp.wait()
    plsc.subcore_barrier()

    lane = lax.broadcasted_iota(jnp.int32, (L,), 0)

    def issue(c):
        k = c % NBUF
        sl = pl.ds(c * CHUNK, CHUNK)
        pltpu.sync_copy(wrel_sh.at[ridx_v.at[sl]], rbufs[k])
        return [
            pltpu.async_copy(embed_hbm.at[hidx_v.at[sl]], hbufs[k], sems[k]),
            pltpu.async_copy(embed_hbm.at[tidx_v.at[sl]], tbufs[k], sems[k]),
        ]

    pending = {}
    for c in range(NBUF - 1):
        pending[c] = issue(c)

    for c in range(NCHUNKS):
        if c + NBUF - 1 < NCHUNKS:
            pending[c + NBUF - 1] = issue(c + NBUF - 1)
        with jax.named_scope("dma_wait%d" % c):
            for cp in pending.pop(c):
                cp.wait()
        k = c % NBUF
        h_v, r_v, t_v = hbufs[k], rbufs[k], tbufs[k]

        def group_body(g, _):
            rows = g * L + lane

            def dstep(_, carry):
                accs, col = carry
                accs = list(accs)
                for u in range(UNROLL):
                    hv = plsc.load_gather(h_v, [rows, col])
                    rv = plsc.load_gather(r_v, [rows, col])
                    tv = plsc.load_gather(t_v, [rows, col])
                    accs[u % NACC] = accs[u % NACC] + hv * rv * tv
                    col = (col + 1) & (FEAT_DIM - 1)
                return tuple(accs), col

            zero = jnp.zeros((L,), jnp.float32)
            accs, _ = lax.fori_loop(
                0, FEAT_DIM // UNROLL, dstep,
                ((zero,) * NACC, lane))
            acc = (accs[0] + accs[1]) + (accs[2] + accs[3])
            out_v[pl.ds(c * CHUNK + g * L, L)] = acc
            return 0

        with jax.named_scope("compute%d" % c):
            lax.fori_loop(0, GROUPS, group_body, 0)

    with jax.named_scope("drain_out"):
        pltpu.sync_copy(out_v, out_hbm.at[pl.ds(base, ROWS_PER_W)])


@jax.jit
def _dist_mult(embed, head, rel, tail, w_relations):
    mesh = plsc.VectorSubcoreMesh(core_axis_name="c", subcore_axis_name="s")
    rowbuf = pltpu.VMEM((CHUNK, FEAT_DIM), jnp.float32)
    run = pl.kernel(
        _sc_body,
        out_type=jax.ShapeDtypeStruct((BATCH,), jnp.float32),
        mesh=mesh,
        compiler_params=pltpu.CompilerParams(needs_layout_passes=False),
        scratch_types=[
            pltpu.VMEM((ROWS_PER_W,), jnp.int32),
            pltpu.VMEM((ROWS_PER_W,), jnp.int32),
            pltpu.VMEM((ROWS_PER_W,), jnp.int32),
            [rowbuf] * NBUF,
            [rowbuf] * NBUF,
            [rowbuf] * NBUF,
            pltpu.VMEM((ROWS_PER_W,), jnp.float32),
            pltpu.VMEM_SHARED((NUM_RELS, FEAT_DIM), jnp.float32),
            [pltpu.SemaphoreType.DMA] * NBUF,
        ],
    )
    return run(embed, head, rel, tail, w_relations)


def kernel(embed, head, rel, tail, w_relations):
    head = head.astype(jnp.int32)
    rel = rel.astype(jnp.int32)
    tail = tail.astype(jnp.int32)
    return _dist_mult(embed, head, rel, tail, w_relations)


# speedup vs baseline: 1.0362x; 1.0046x over previous
import jax
import jax.numpy as jnp
from jax import lax
from jax.experimental import pallas as pl
from jax.experimental.pallas import tpu as pltpu
from jax.experimental.pallas import tpu_sc as plsc

NUM_ENTITIES = 100000
NUM_RELS = 1000
FEAT_DIM = 128
BATCH = 16384

NC = 2
NS = 16
L = 16
NW = NC * NS
ROWS_PER_W = BATCH // NW
CHUNK = 64
NCHUNKS = ROWS_PER_W // CHUNK
NBUF = 4
GROUPS = CHUNK // L
UNROLL = 16
NACC = 4


def _sc_body(embed_hbm, head_hbm, rel_hbm, tail_hbm, wrel_hbm, out_hbm,
             hidx_v, ridx_v, tidx_v, hbufs, rbufs, tbufs, out_v, wrel_sh,
             sems, rsems):
    sid = lax.axis_index("s")
    wid = sid * NC + lax.axis_index("c")
    base = wid * ROWS_PER_W

    @pl.when(sid == 0)
    def _():
        pltpu.sync_copy(wrel_hbm, wrel_sh)

    with jax.named_scope("stage_idx"):
        cp_i = [
            pltpu.async_copy(head_hbm.at[pl.ds(base, ROWS_PER_W)], hidx_v, sems[0]),
            pltpu.async_copy(rel_hbm.at[pl.ds(base, ROWS_PER_W)], ridx_v, sems[1]),
            pltpu.async_copy(tail_hbm.at[pl.ds(base, ROWS_PER_W)], tidx_v, sems[2]),
        ]
        for cp in cp_i:
            cp.wait()
    plsc.subcore_barrier()

    lane = lax.broadcasted_iota(jnp.int32, (L,), 0)

    def issue(c):
        k = c % NBUF
        sl = pl.ds(c * CHUNK, CHUNK)
        return [
            pltpu.async_copy(embed_hbm.at[hidx_v.at[sl]], hbufs[k], sems[k]),
            pltpu.async_copy(embed_hbm.at[tidx_v.at[sl]], tbufs[k], sems[k]),
            pltpu.async_copy(wrel_sh.at[ridx_v.at[sl]], rbufs[k], rsems[k]),
        ]

    pending = {}
    for c in range(NBUF - 1):
        pending[c] = issue(c)

    for c in range(NCHUNKS):
        if c + NBUF - 1 < NCHUNKS:
            pending[c + NBUF - 1] = issue(c + NBUF - 1)
        with jax.named_scope("dma_wait%d" % c):
            for cp in pending.pop(c):
                cp.wait()
        k = c % NBUF
        h_v, r_v, t_v = hbufs[k], rbufs[k], tbufs[k]

        def group_body(g, _):
            rows = g * L + lane

            def dstep(_, carry):
                accs, col = carry
                accs = list(accs)
                for u in range(UNROLL):
                    hv = plsc.load_gather(h_v, [rows, col])
                    rv = plsc.load_gather(r_v, [rows, col])
                    tv = plsc.load_gather(t_v, [rows, col])
                    accs[u % NACC] = accs[u % NACC] + hv * rv * tv
                    col = (col + 1) & (FEAT_DIM - 1)
                return tuple(accs), col

            zero = jnp.zeros((L,), jnp.float32)
            accs, _ = lax.fori_loop(
                0, FEAT_DIM // UNROLL, dstep,
                ((zero,) * NACC, lane))
            acc = (accs[0] + accs[1]) + (accs[2] + accs[3])
            out_v[pl.ds(c * CHUNK + g * L, L)] = acc
            return 0

        with jax.named_scope("compute%d" % c):
            lax.fori_loop(0, GROUPS, group_body, 0)

    with jax.named_scope("drain_out"):
        pltpu.sync_copy(out_v, out_hbm.at[pl.ds(base, ROWS_PER_W)])


@jax.jit
def _dist_mult(embed, head, rel, tail, w_relations):
    mesh = plsc.VectorSubcoreMesh(core_axis_name="c", subcore_axis_name="s")
    rowbuf = pltpu.VMEM((CHUNK, FEAT_DIM), jnp.float32)
    run = pl.kernel(
        _sc_body,
        out_type=jax.ShapeDtypeStruct((BATCH,), jnp.float32),
        mesh=mesh,
        compiler_params=pltpu.CompilerParams(needs_layout_passes=False),
        scratch_types=[
            pltpu.VMEM((ROWS_PER_W,), jnp.int32),
            pltpu.VMEM((ROWS_PER_W,), jnp.int32),
            pltpu.VMEM((ROWS_PER_W,), jnp.int32),
            [rowbuf] * NBUF,
            [rowbuf] * NBUF,
            [rowbuf] * NBUF,
            pltpu.VMEM((ROWS_PER_W,), jnp.float32),
            pltpu.VMEM_SHARED((NUM_RELS, FEAT_DIM), jnp.float32),
            [pltpu.SemaphoreType.DMA] * NBUF,
            [pltpu.SemaphoreType.DMA] * NBUF,
        ],
    )
    return run(embed, head, rel, tail, w_relations)


def kernel(embed, head, rel, tail, w_relations):
    head = head.astype(jnp.int32)
    rel = rel.astype(jnp.int32)
    tail = tail.astype(jnp.int32)
    return _dist_mult(embed, head, rel, tail, w_relations)


# speedup vs baseline: 1.0373x; 1.0010x over previous
import jax
import jax.numpy as jnp
from jax import lax
from jax.experimental import pallas as pl
from jax.experimental.pallas import tpu as pltpu
from jax.experimental.pallas import tpu_sc as plsc

NUM_ENTITIES = 100000
NUM_RELS = 1000
FEAT_DIM = 128
BATCH = 16384

NC = 2
NS = 16
L = 16
NW = NC * NS
ROWS_PER_W = BATCH // NW
CHUNK = 64
NCHUNKS = ROWS_PER_W // CHUNK
NBUF = 4
GROUPS = CHUNK // L
UNROLL = 16
NACC = 4


def _sc_body(embed_hbm, head_hbm, rel_hbm, tail_hbm, wrel_hbm, out_hbm,
             hidx_v, ridx_v, tidx_v, hbufs, rbufs, tbufs, out_v, wrel_sh,
             sems, rsems):
    sid = lax.axis_index("s")
    wid = sid * NC + lax.axis_index("c")
    base = wid * ROWS_PER_W

    @pl.when(sid == 0)
    def _():
        pltpu.sync_copy(wrel_hbm, wrel_sh)

    with jax.named_scope("stage_idx"):
        cp_i = [
            pltpu.async_copy(head_hbm.at[pl.ds(base, ROWS_PER_W)], hidx_v, sems[0]),
            pltpu.async_copy(rel_hbm.at[pl.ds(base, ROWS_PER_W)], ridx_v, sems[1]),
            pltpu.async_copy(tail_hbm.at[pl.ds(base, ROWS_PER_W)], tidx_v, sems[2]),
        ]
        for cp in cp_i:
            cp.wait()
    plsc.subcore_barrier()

    lane = lax.broadcasted_iota(jnp.int32, (L,), 0)

    def issue(c):
        k = c % NBUF
        sl = pl.ds(c * CHUNK, CHUNK)
        return [
            pltpu.async_copy(embed_hbm.at[hidx_v.at[sl]], hbufs[k], sems[k]),
            pltpu.async_copy(embed_hbm.at[tidx_v.at[sl]], tbufs[k], sems[k]),
            pltpu.async_copy(wrel_sh.at[ridx_v.at[sl]], rbufs[k], rsems[k]),
        ]

    pending = {0: issue(0)}

    for c in range(NCHUNKS):
        with jax.named_scope("dma_wait%d" % c):
            for cp in pending.pop(c):
                cp.wait()
        if c == 0:
            for cc in range(1, NBUF - 1):
                pending[cc] = issue(cc)
        if c + NBUF - 1 < NCHUNKS:
            pending[c + NBUF - 1] = issue(c + NBUF - 1)
        k = c % NBUF
        h_v, r_v, t_v = hbufs[k], rbufs[k], tbufs[k]

        def group_body(g, _):
            rows = g * L + lane

            def dstep(_, carry):
                accs, col = carry
                accs = list(accs)
                for u in range(UNROLL):
                    hv = plsc.load_gather(h_v, [rows, col])
                    rv = plsc.load_gather(r_v, [rows, col])
                    tv = plsc.load_gather(t_v, [rows, col])
                    accs[u % NACC] = accs[u % NACC] + hv * rv * tv
                    col = (col + 1) & (FEAT_DIM - 1)
                return tuple(accs), col

            zero = jnp.zeros((L,), jnp.float32)
            accs, _ = lax.fori_loop(
                0, FEAT_DIM // UNROLL, dstep,
                ((zero,) * NACC, lane))
            acc = (accs[0] + accs[1]) + (accs[2] + accs[3])
            out_v[pl.ds(c * CHUNK + g * L, L)] = acc
            return 0

        with jax.named_scope("compute%d" % c):
            lax.fori_loop(0, GROUPS, group_body, 0)

    with jax.named_scope("drain_out"):
        pltpu.sync_copy(out_v, out_hbm.at[pl.ds(base, ROWS_PER_W)])


@jax.jit
def _dist_mult(embed, head, rel, tail, w_relations):
    mesh = plsc.VectorSubcoreMesh(core_axis_name="c", subcore_axis_name="s")
    rowbuf = pltpu.VMEM((CHUNK, FEAT_DIM), jnp.float32)
    run = pl.kernel(
        _sc_body,
        out_type=jax.ShapeDtypeStruct((BATCH,), jnp.float32),
        mesh=mesh,
        compiler_params=pltpu.CompilerParams(needs_layout_passes=False),
        scratch_types=[
            pltpu.VMEM((ROWS_PER_W,), jnp.int32),
            pltpu.VMEM((ROWS_PER_W,), jnp.int32),
            pltpu.VMEM((ROWS_PER_W,), jnp.int32),
            [rowbuf] * NBUF,
            [rowbuf] * NBUF,
            [rowbuf] * NBUF,
            pltpu.VMEM((ROWS_PER_W,), jnp.float32),
            pltpu.VMEM_SHARED((NUM_RELS, FEAT_DIM), jnp.float32),
            [pltpu.SemaphoreType.DMA] * NBUF,
            [pltpu.SemaphoreType.DMA] * NBUF,
        ],
    )
    return run(embed, head, rel, tail, w_relations)


def kernel(embed, head, rel, tail, w_relations):
    head = head.astype(jnp.int32)
    rel = rel.astype(jnp.int32)
    tail = tail.astype(jnp.int32)
    return _dist_mult(embed, head, rel, tail, w_relations)


# speedup vs baseline: 1.1129x; 1.0729x over previous
import jax
import jax.numpy as jnp
from jax import lax
from jax.experimental import pallas as pl
from jax.experimental.pallas import tpu as pltpu
from jax.experimental.pallas import tpu_sc as plsc

NUM_ENTITIES = 100000
NUM_RELS = 1000
FEAT_DIM = 128
BATCH = 16384

NC = 2
NS = 16
L = 16
NW = NC * NS
ROWS_PER_W = BATCH // NW
CHUNK = 64
NCHUNKS = ROWS_PER_W // CHUNK
NBUF = 4
GROUPS = CHUNK // L
UNROLL = 16
NACC = 4


def _sc_body(embed_hbm, head_hbm, rel_hbm, tail_hbm, wrel_hbm, out_hbm,
             hidx_v, ridx_v, tidx_v, hbufs, rbufs, tbufs, out_v, wrel_sh,
             sems, rsems):
    sid = lax.axis_index("s")
    wid = sid * NC + lax.axis_index("c")
    base = wid * ROWS_PER_W

    @pl.when(sid == 0)
    def _():
        pltpu.async_copy(wrel_hbm, wrel_sh, rsems[0])

    with jax.named_scope("stage_idx"):
        cp_i = [
            pltpu.async_copy(head_hbm.at[pl.ds(base, ROWS_PER_W)], hidx_v, sems[0]),
            pltpu.async_copy(rel_hbm.at[pl.ds(base, ROWS_PER_W)], ridx_v, sems[1]),
            pltpu.async_copy(tail_hbm.at[pl.ds(base, ROWS_PER_W)], tidx_v, sems[2]),
        ]
        for cp in cp_i:
            cp.wait()

    lane = lax.broadcasted_iota(jnp.int32, (L,), 0)

    def issue_ht(c):
        k = c % NBUF
        sl = pl.ds(c * CHUNK, CHUNK)
        return [
            pltpu.async_copy(embed_hbm.at[hidx_v.at[sl]], hbufs[k], sems[k]),
            pltpu.async_copy(embed_hbm.at[tidx_v.at[sl]], tbufs[k], sems[k]),
        ]

    def issue_r(c):
        k = c % NBUF
        sl = pl.ds(c * CHUNK, CHUNK)
        return pltpu.async_copy(wrel_sh.at[ridx_v.at[sl]], rbufs[k], rsems[k])

    pend_ht = {0: issue_ht(0), 1: issue_ht(1)}
    @pl.when(sid == 0)
    def _():
        pltpu.make_async_copy(wrel_hbm, wrel_sh, rsems[0]).wait()
    plsc.subcore_barrier()
    pend_r = {0: issue_r(0), 1: issue_r(1)}

    for c in range(NCHUNKS):
        with jax.named_scope("dma_wait%d" % c):
            for cp in pend_ht.pop(c):
                cp.wait()
            pend_r.pop(c).wait()
        if c + 2 < NCHUNKS:
            pend_ht[c + 2] = issue_ht(c + 2)
            pend_r[c + 2] = issue_r(c + 2)
        k = c % NBUF
        h_v, r_v, t_v = hbufs[k], rbufs[k], tbufs[k]

        def group_body(g, _):
            rows = g * L + lane

            def dstep(_, carry):
                accs, col = carry
                accs = list(accs)
                for u in range(UNROLL):
                    hv = plsc.load_gather(h_v, [rows, col])
                    rv = plsc.load_gather(r_v, [rows, col])
                    tv = plsc.load_gather(t_v, [rows, col])
                    accs[u % NACC] = accs[u % NACC] + hv * rv * tv
                    col = (col + 1) & (FEAT_DIM - 1)
                return tuple(accs), col

            zero = jnp.zeros((L,), jnp.float32)
            accs, _ = lax.fori_loop(
                0, FEAT_DIM // UNROLL, dstep,
                ((zero,) * NACC, lane))
            acc = (accs[0] + accs[1]) + (accs[2] + accs[3])
            out_v[pl.ds(c * CHUNK + g * L, L)] = acc
            return 0

        with jax.named_scope("compute%d" % c):
            lax.fori_loop(0, GROUPS, group_body, 0)

    with jax.named_scope("drain_out"):
        pltpu.sync_copy(out_v, out_hbm.at[pl.ds(base, ROWS_PER_W)])


@jax.jit
def _dist_mult(embed, head, rel, tail, w_relations):
    mesh = plsc.VectorSubcoreMesh(core_axis_name="c", subcore_axis_name="s")
    rowbuf = pltpu.VMEM((CHUNK, FEAT_DIM), jnp.float32)
    run = pl.kernel(
        _sc_body,
        out_type=jax.ShapeDtypeStruct((BATCH,), jnp.float32),
        mesh=mesh,
        compiler_params=pltpu.CompilerParams(needs_layout_passes=False),
        scratch_types=[
            pltpu.VMEM((ROWS_PER_W,), jnp.int32),
            pltpu.VMEM((ROWS_PER_W,), jnp.int32),
            pltpu.VMEM((ROWS_PER_W,), jnp.int32),
            [rowbuf] * NBUF,
            [rowbuf] * NBUF,
            [rowbuf] * NBUF,
            pltpu.VMEM((ROWS_PER_W,), jnp.float32),
            pltpu.VMEM_SHARED((NUM_RELS, FEAT_DIM), jnp.float32),
            [pltpu.SemaphoreType.DMA] * NBUF,
            [pltpu.SemaphoreType.DMA] * NBUF,
        ],
    )
    return run(embed, head, rel, tail, w_relations)


def kernel(embed, head, rel, tail, w_relations):
    head = head.astype(jnp.int32)
    rel = rel.astype(jnp.int32)
    tail = tail.astype(jnp.int32)
    return _dist_mult(embed, head, rel, tail, w_relations)


# speedup vs baseline: 1.1211x; 1.0073x over previous
import jax
import jax.numpy as jnp
from jax import lax
from jax.experimental import pallas as pl
from jax.experimental.pallas import tpu as pltpu
from jax.experimental.pallas import tpu_sc as plsc

NUM_ENTITIES = 100000
NUM_RELS = 1000
FEAT_DIM = 128
BATCH = 16384

NC = 2
NS = 16
L = 16
NW = NC * NS
ROWS_PER_W = BATCH // NW
CHUNK = 64
NCHUNKS = ROWS_PER_W // CHUNK
NBUF = 3
GROUPS = CHUNK // L
UNROLL = 16
NACC = 4


def _sc_body(embed_hbm, head_hbm, rel_hbm, tail_hbm, wrel_hbm, out_hbm,
             hidx_v, ridx_v, tidx_v, hbufs, rbufs, tbufs, out_v, wrel_sh,
             sems, rsems):
    sid = lax.axis_index("s")
    wid = sid * NC + lax.axis_index("c")
    base = wid * ROWS_PER_W

    @pl.when(sid == 0)
    def _():
        pltpu.async_copy(wrel_hbm, wrel_sh, rsems[0])

    with jax.named_scope("stage_idx"):
        cp_i = [
            pltpu.async_copy(head_hbm.at[pl.ds(base, ROWS_PER_W)], hidx_v, sems[0]),
            pltpu.async_copy(rel_hbm.at[pl.ds(base, ROWS_PER_W)], ridx_v, sems[1]),
            pltpu.async_copy(tail_hbm.at[pl.ds(base, ROWS_PER_W)], tidx_v, sems[2]),
        ]
        for cp in cp_i:
            cp.wait()

    lane = lax.broadcasted_iota(jnp.int32, (L,), 0)

    def issue_ht(c):
        k = c % NBUF
        sl = pl.ds(c * CHUNK, CHUNK)
        return [
            pltpu.async_copy(embed_hbm.at[hidx_v.at[sl]], hbufs[k], sems[k]),
            pltpu.async_copy(embed_hbm.at[tidx_v.at[sl]], tbufs[k], sems[k]),
        ]

    def issue_r(c):
        sl = pl.ds(c * CHUNK, CHUNK)
        return pltpu.async_copy(wrel_sh.at[ridx_v.at[sl]], rbufs[c], rsems[c])

    pend_ht = {0: issue_ht(0), 1: issue_ht(1)}
    @pl.when(sid == 0)
    def _():
        pltpu.make_async_copy(wrel_hbm, wrel_sh, rsems[0]).wait()
    plsc.subcore_barrier()
    pend_r = [issue_r(c) for c in range(NCHUNKS)]

    for c in range(NCHUNKS):
        with jax.named_scope("dma_wait%d" % c):
            for cp in pend_ht.pop(c):
                cp.wait()
            pend_r[c].wait()
        if c + 2 < NCHUNKS:
            pend_ht[c + 2] = issue_ht(c + 2)
        k = c % NBUF
        h_v, r_v, t_v = hbufs[k], rbufs[c], tbufs[k]

        def group_body(g, _):
            rows = g * L + lane

            def dstep(_, carry):
                accs, col = carry
                accs = list(accs)
                for u in range(UNROLL):
                    hv = plsc.load_gather(h_v, [rows, col])
                    rv = plsc.load_gather(r_v, [rows, col])
                    tv = plsc.load_gather(t_v, [rows, col])
                    accs[u % NACC] = accs[u % NACC] + hv * rv * tv
                    col = (col + 1) & (FEAT_DIM - 1)
                return tuple(accs), col

            zero = jnp.zeros((L,), jnp.float32)
            accs, _ = lax.fori_loop(
                0, FEAT_DIM // UNROLL, dstep,
                ((zero,) * NACC, lane))
            acc = (accs[0] + accs[1]) + (accs[2] + accs[3])
            out_v[pl.ds(c * CHUNK + g * L, L)] = acc
            return 0

        with jax.named_scope("compute%d" % c):
            lax.fori_loop(0, GROUPS, group_body, 0)

    with jax.named_scope("drain_out"):
        pltpu.sync_copy(out_v, out_hbm.at[pl.ds(base, ROWS_PER_W)])


@jax.jit
def _dist_mult(embed, head, rel, tail, w_relations):
    mesh = plsc.VectorSubcoreMesh(core_axis_name="c", subcore_axis_name="s")
    rowbuf = pltpu.VMEM((CHUNK, FEAT_DIM), jnp.float32)
    run = pl.kernel(
        _sc_body,
        out_type=jax.ShapeDtypeStruct((BATCH,), jnp.float32),
        mesh=mesh,
        compiler_params=pltpu.CompilerParams(needs_layout_passes=False),
        scratch_types=[
            pltpu.VMEM((ROWS_PER_W,), jnp.int32),
            pltpu.VMEM((ROWS_PER_W,), jnp.int32),
            pltpu.VMEM((ROWS_PER_W,), jnp.int32),
            [rowbuf] * NBUF,
            [rowbuf] * NCHUNKS,
            [rowbuf] * NBUF,
            pltpu.VMEM((ROWS_PER_W,), jnp.float32),
            pltpu.VMEM_SHARED((NUM_RELS, FEAT_DIM), jnp.float32),
            [pltpu.SemaphoreType.DMA] * NBUF,
            [pltpu.SemaphoreType.DMA] * NCHUNKS,
        ],
    )
    return run(embed, head, rel, tail, w_relations)


def kernel(embed, head, rel, tail, w_relations):
    head = head.astype(jnp.int32)
    rel = rel.astype(jnp.int32)
    tail = tail.astype(jnp.int32)
    return _dist_mult(embed, head, rel, tail, w_relations)
